# baseline (speedup 1.0000x reference)
import functools

import jax
import jax.numpy as jnp
from jax import lax
from jax.experimental import pallas as pl
from jax.experimental.pallas import tpu as pltpu
from jax.experimental.pallas import tpu_sc as plsc

B = 16384
D = 16
NC = 2
NS = 16
NW = NC * NS
BPW = B // NW
CHUNK = 128
NCHUNK = BPW // CHUNK

_LOG1P_COEFS = (
    5.2394028005e-09,
    9.9999891058e-01,
    -4.9996224452e-01,
    3.3281842540e-01,
    -2.4635660618e-01,
    1.8468848463e-01,
    -1.2526661441e-01,
    6.6512479382e-02,
    -2.3038279977e-02,
    3.7526242255e-03,
)


def _softplus(x):
    t = jnp.exp(-jnp.abs(x))
    p = jnp.full((16,), _LOG1P_COEFS[-1], jnp.float32)
    for c in reversed(_LOG1P_COEFS[:-1]):
        p = p * t + jnp.float32(c)
    return jnp.maximum(x, jnp.float32(0.0)) + p


def _mirt_body(user_hbm, item_hbm, theta_hbm, a_hbm, b_hbm, out_hbm,
               uidx_v, iidx_v, th_v, a_v, b_v, out_v, sem_t, sem_a, sem_b):
    wid = lax.axis_index("s") * NC + lax.axis_index("c")

    pltpu.sync_copy(user_hbm.at[wid], uidx_v)
    pltpu.sync_copy(item_hbm.at[wid], iidx_v)

    copies = []
    for k in range(NCHUNK):
        copies.append(pltpu.async_copy(theta_hbm.at[uidx_v.at[k]], th_v.at[k], sem_t))
        copies.append(pltpu.async_copy(a_hbm.at[iidx_v.at[k]], a_v.at[k], sem_a))
        copies.append(pltpu.async_copy(b_hbm.at[iidx_v.at[k]], b_v.at[k], sem_b))
    for cp in copies:
        cp.wait()

    iota16 = lax.iota(jnp.int32, 16)

    for k in range(NCHUNK):
        kvec = jnp.full((16,), k, jnp.int32)

        def block(j, carry, k=k, kvec=kvec):
            rows = j * 16 + iota16
            acc = -plsc.load_gather(b_v, [kvec, rows])
            for d in range(D):
                dvec = jnp.full((16,), d, jnp.int32)
                th = plsc.load_gather(th_v, [kvec, rows, dvec])
                av = plsc.load_gather(a_v, [kvec, rows, dvec])
                acc = acc + _softplus(av) * th
            res = jnp.float32(1.0) / (jnp.float32(1.0) + jnp.exp(-acc))
            out_v[pl.ds(k * CHUNK + j * 16, 16)] = res
            return carry

        lax.fori_loop(0, CHUNK // 16, block, 0)

    pltpu.sync_copy(out_v, out_hbm.at[wid])


_mirt = functools.partial(
    pl.kernel,
    out_type=jax.ShapeDtypeStruct((NW, BPW), jnp.float32),
    mesh=plsc.VectorSubcoreMesh(core_axis_name="c", subcore_axis_name="s"),
    compiler_params=pltpu.CompilerParams(
        needs_layout_passes=False, use_tc_tiling_on_sc=False),
    scratch_types=[
        pltpu.VMEM((NCHUNK, CHUNK), jnp.int32),
        pltpu.VMEM((NCHUNK, CHUNK), jnp.int32),
        pltpu.VMEM((NCHUNK, CHUNK, D), jnp.float32),
        pltpu.VMEM((NCHUNK, CHUNK, D), jnp.float32),
        pltpu.VMEM((NCHUNK, CHUNK), jnp.float32),
        pltpu.VMEM((BPW,), jnp.float32),
        pltpu.SemaphoreType.DMA,
        pltpu.SemaphoreType.DMA,
        pltpu.SemaphoreType.DMA,
    ],
)(_mirt_body)


def kernel(user, item, theta_table, a_table, b_table):
    user = user.astype(jnp.int32).reshape(NW, NCHUNK, CHUNK)
    item = item.astype(jnp.int32).reshape(NW, NCHUNK, CHUNK)
    b_flat = b_table.reshape(-1)
    out = _mirt(user, item, theta_table, a_table, b_flat)
    return out.reshape(B)
